# baseline (speedup 1.0000x reference)
import jax
import jax.numpy as jnp
from jax.experimental import pallas as pl
from jax.experimental.pallas import tpu as pltpu

_WIDE = 1024


def _mlp3_kernel(x_ref, w1_ref, b1_ref, w2_ref, b2_ref, w3_ref, b3_ref,
                 o_ref, w1b, w2b, xb, h1b, h2b0):
    n = b1_ref.shape[1]
    m = xb.shape[0]
    o = b3_ref.shape[1]
    tnc = w1_ref.shape[1]
    tmx = x_ref.shape[0]
    nc = n // tnc
    tm = tmx
    nrows = m // tm
    s = pl.program_id(0)

    for _rb in range(m // tmx):
        @pl.when(s == _rb)
        def _cast_x(_rb=_rb):
            xb[_rb * tmx:(_rb + 1) * tmx, :] = x_ref[...].astype(jnp.bfloat16)

    @pl.when(s < nc)
    def _stream_w1_chunk():
        sl = pl.ds(s * tnc, tnc)
        wc = w1_ref[...].astype(jnp.bfloat16)
        w1b[:, sl] = wc
        acc = jnp.dot(xb[pl.ds(0, tm), :], wc,
                      preferred_element_type=jnp.float32)
        h1b[pl.ds(0, tm), sl] = jnp.maximum(
            acc + b1_ref[:, sl], 0.0).astype(jnp.bfloat16)

    @pl.when((s >= nc) & (s < 2 * nc))
    def _stream_w2_chunk():
        c = s - nc
        sl = pl.ds(c * tnc, tnc)
        wc = w2_ref[...].astype(jnp.bfloat16)
        w2b[:, sl] = wc
        acc = jnp.dot(h1b[pl.ds(0, tm), :], wc,
                      preferred_element_type=jnp.float32)
        h2b0[:, sl] = jnp.maximum(
            acc + b2_ref[:, sl], 0.0).astype(jnp.bfloat16)

    @pl.when(s == 2 * nc - 1)
    def _row0_out():
        w3c = w3_ref[...].astype(jnp.bfloat16)
        y = jnp.dot(h2b0[...], w3c, preferred_element_type=jnp.float32)
        o_ref[pl.ds(0, tm), :] = y + b3_ref[...]

    @pl.when(s == 2 * nc)
    def _later_rows():
        wide = min(_WIDE, n)
        for c in range(n // wide):
            sl = slice(c * wide, (c + 1) * wide)
            for r in range(1, nrows):
                rs = slice(r * tm, (r + 1) * tm)
                acc = jnp.dot(xb[rs, :], w1b[:, sl],
                              preferred_element_type=jnp.float32)
                h1b[rs, sl] = jnp.maximum(
                    acc + b1_ref[:, sl], 0.0).astype(jnp.bfloat16)
        for r in range(1, nrows):
            o_ref[r * tm:(r + 1) * tm, :] = jnp.broadcast_to(
                b3_ref[...], (tm, o))
        w3c = w3_ref[...].astype(jnp.bfloat16)
        for c in range(n // wide):
            sl = slice(c * wide, (c + 1) * wide)
            for r in range(1, nrows):
                rs = slice(r * tm, (r + 1) * tm)
                acc = jnp.dot(h1b[rs, :], w2b[:, sl],
                              preferred_element_type=jnp.float32)
                h2c = jnp.maximum(
                    acc + b2_ref[:, sl], 0.0).astype(jnp.bfloat16)
                o_ref[rs, :] += jnp.dot(
                    h2c, w3c[sl, :], preferred_element_type=jnp.float32)


def _mlp3(x, w1, b1r, w2, b2r, w3, b3r, *, tmx, tnc):
    M, K = x.shape
    N = w1.shape[1]
    O = w3.shape[1]
    nc = N // tnc
    nsteps = 2 * nc + 1
    flops = 2 * M * K * N + 2 * M * N * N + 2 * M * N * O
    bytes_accessed = 4 * (M * K + K * N + N * N + N * O + M * O)

    return pl.pallas_call(
        _mlp3_kernel,
        out_shape=jax.ShapeDtypeStruct((M, O), jnp.float32),
        grid=(nsteps,),
        in_specs=[
            pl.BlockSpec((tmx, K), lambda s: (jnp.minimum(s, M // tmx - 1), 0)),
            pl.BlockSpec((K, tnc), lambda s: (0, jnp.minimum(s, nc - 1))),
            pl.BlockSpec((1, N), lambda s: (0, 0)),
            pl.BlockSpec((K, tnc),
                         lambda s: (0, jnp.clip(s - nc, 0, nc - 1))),
            pl.BlockSpec((1, N), lambda s: (0, 0)),
            pl.BlockSpec((N, O), lambda s: (0, 0)),
            pl.BlockSpec((1, O), lambda s: (0, 0)),
        ],
        out_specs=pl.BlockSpec((M, O), lambda s: (0, 0)),
        scratch_shapes=[
            pltpu.VMEM((K, N), jnp.bfloat16),
            pltpu.VMEM((N, N), jnp.bfloat16),
            pltpu.VMEM((M, K), jnp.bfloat16),
            pltpu.VMEM((M, N), jnp.bfloat16),
            pltpu.VMEM((tmx, N), jnp.bfloat16),
        ],
        compiler_params=pltpu.CompilerParams(
            dimension_semantics=("arbitrary",),
        ),
        cost_estimate=pl.CostEstimate(
            flops=flops, transcendentals=0, bytes_accessed=bytes_accessed
        ),
    )(x, w1, b1r, w2, b2r, w3, b3r)


@jax.jit
def kernel(x, w1, b1, w2, b2, w3, b3):
    M = x.shape[0]
    N = w1.shape[1]
    O = w3.shape[1]
    tnc = min(512, max(N // 2, 128))
    nc = N // tnc
    tmx = max(M // (2 * nc), 8)
    return _mlp3(x, w1, b1.reshape(1, N), w2, b2.reshape(1, N),
                 w3, b3.reshape(1, O), tmx=tmx, tnc=tnc)


# speedup vs baseline: 1.0689x; 1.0689x over previous
import jax
import jax.numpy as jnp
from jax.experimental import pallas as pl
from jax.experimental.pallas import tpu as pltpu

_WIDE = 2048


def _mlp3_kernel(x_ref, w1_ref, b1_ref, w2_ref, b2_ref, w3_ref, b3_ref,
                 o_ref, w1b, w2b, xb, h1b, h2b):
    n = b1_ref.shape[1]
    tnc = w1_ref.shape[1]
    nc = n // tnc
    s = pl.program_id(0)

    def finish(h2full):
        w3c = w3_ref[...].astype(jnp.bfloat16)
        y = jnp.dot(h2full, w3c, preferred_element_type=jnp.float32)
        o_ref[...] = y + b3_ref[...]

    @pl.when(s == 0)
    def _cast_x0():
        xb[...] = x_ref[...].astype(jnp.bfloat16)

    @pl.when(s < nc)
    def _stream_w1_chunk():
        sl = pl.ds(s * tnc, tnc)
        wc = w1_ref[...].astype(jnp.bfloat16)
        w1b[:, sl] = wc
        acc = jnp.dot(xb[...], wc, preferred_element_type=jnp.float32)
        h1b[:, sl] = jnp.maximum(acc + b1_ref[:, sl], 0.0).astype(jnp.bfloat16)

    @pl.when((s >= nc) & (s < 2 * nc))
    def _stream_w2_chunk():
        c = s - nc
        sl = pl.ds(c * tnc, tnc)
        wc = w2_ref[...].astype(jnp.bfloat16)
        w2b[:, sl] = wc
        acc = jnp.dot(h1b[...], wc, preferred_element_type=jnp.float32)
        h2b[:, sl] = jnp.maximum(acc + b2_ref[:, sl], 0.0).astype(jnp.bfloat16)

    @pl.when(s == 2 * nc - 1)
    def _row0_out():
        finish(h2b[...])

    @pl.when(s >= 2 * nc)
    def _later_rows():
        wide = min(_WIDE, n)
        xr = x_ref[...].astype(jnp.bfloat16)
        for c in range(n // wide):
            sl = pl.ds(c * wide, wide)
            acc = jnp.dot(xr, w1b[:, sl], preferred_element_type=jnp.float32)
            h1b[:, sl] = jnp.maximum(
                acc + b1_ref[:, sl], 0.0).astype(jnp.bfloat16)
        for c in range(n // wide):
            sl = pl.ds(c * wide, wide)
            acc = jnp.dot(h1b[...], w2b[:, sl],
                          preferred_element_type=jnp.float32)
            h2b[:, sl] = jnp.maximum(
                acc + b2_ref[:, sl], 0.0).astype(jnp.bfloat16)
        finish(h2b[...])


def _mlp3(x, w1, b1r, w2, b2r, w3, b3r, *, tm, tnc):
    M, K = x.shape
    N = w1.shape[1]
    O = w3.shape[1]
    nc = N // tnc
    nrows = M // tm
    nsteps = 2 * nc + (nrows - 1)
    flops = 2 * M * K * N + 2 * M * N * N + 2 * M * N * O
    bytes_accessed = 4 * (M * K + K * N + N * N + N * O + M * O)

    row_of = lambda s: jnp.maximum(s - (2 * nc - 1), 0)
    return pl.pallas_call(
        _mlp3_kernel,
        out_shape=jax.ShapeDtypeStruct((M, O), jnp.float32),
        grid=(nsteps,),
        in_specs=[
            pl.BlockSpec((tm, K), lambda s: (row_of(s), 0)),
            pl.BlockSpec((K, tnc), lambda s: (0, jnp.minimum(s, nc - 1))),
            pl.BlockSpec((1, N), lambda s: (0, 0)),
            pl.BlockSpec((K, tnc),
                         lambda s: (0, jnp.clip(s - nc, 0, nc - 1))),
            pl.BlockSpec((1, N), lambda s: (0, 0)),
            pl.BlockSpec((N, O), lambda s: (0, 0)),
            pl.BlockSpec((1, O), lambda s: (0, 0)),
        ],
        out_specs=pl.BlockSpec((tm, O), lambda s: (row_of(s), 0)),
        scratch_shapes=[
            pltpu.VMEM((K, N), jnp.bfloat16),
            pltpu.VMEM((N, N), jnp.bfloat16),
            pltpu.VMEM((tm, K), jnp.bfloat16),
            pltpu.VMEM((tm, N), jnp.bfloat16),
            pltpu.VMEM((tm, N), jnp.bfloat16),
        ],
        compiler_params=pltpu.CompilerParams(
            dimension_semantics=("arbitrary",),
        ),
        cost_estimate=pl.CostEstimate(
            flops=flops, transcendentals=0, bytes_accessed=bytes_accessed
        ),
    )(x, w1, b1r, w2, b2r, w3, b3r)


@jax.jit
def kernel(x, w1, b1, w2, b2, w3, b3):
    M = x.shape[0]
    N = w1.shape[1]
    O = w3.shape[1]
    tm = min(512, max(M // 4, 8))
    tnc = min(512, max(N // 2, 128))
    return _mlp3(x, w1, b1.reshape(1, N), w2, b2.reshape(1, N),
                 w3, b3.reshape(1, O), tm=tm, tnc=tnc)


# speedup vs baseline: 1.0718x; 1.0027x over previous
import jax
import jax.numpy as jnp
from jax.experimental import pallas as pl
from jax.experimental.pallas import tpu as pltpu


def _mlp3_kernel(x_ref, w1_ref, b1_ref, w2_ref, b2_ref, w3_ref, b3_ref,
                 o_ref, w1b, w2b, xb, h1b, h2b):
    n = b1_ref.shape[1]
    tnc = w1_ref.shape[1]
    nc = n // tnc
    s = pl.program_id(0)

    def layer2_and_out(h1src):
        for c in range(nc):
            sl = slice(c * tnc, (c + 1) * tnc)
            acc = jnp.dot(h1src, w2b[:, sl], preferred_element_type=jnp.float32)
            h2b[:, sl] = jnp.maximum(
                acc + b2_ref[:, sl], 0.0).astype(jnp.bfloat16)
        w3c = w3_ref[...].astype(jnp.bfloat16)
        y = jnp.dot(h2b[...], w3c, preferred_element_type=jnp.float32)
        o_ref[...] = y + b3_ref[...]

    @pl.when(s == 0)
    def _cast_x0():
        xb[...] = x_ref[...].astype(jnp.bfloat16)

    @pl.when(s < nc)
    def _stream_chunks():
        sl = pl.ds(s * tnc, tnc)
        w1c = w1_ref[...].astype(jnp.bfloat16)
        w1b[:, sl] = w1c
        w2b[:, sl] = w2_ref[...].astype(jnp.bfloat16)
        acc = jnp.dot(xb[...], w1c, preferred_element_type=jnp.float32)
        h1b[:, sl] = jnp.maximum(acc + b1_ref[:, sl], 0.0).astype(jnp.bfloat16)

    @pl.when(s == nc - 1)
    def _row0_tail():
        layer2_and_out(h1b[...])

    @pl.when(s >= nc)
    def _later_rows():
        xr = x_ref[...].astype(jnp.bfloat16)
        for c in range(nc):
            sl = slice(c * tnc, (c + 1) * tnc)
            acc = jnp.dot(xr, w1b[:, sl], preferred_element_type=jnp.float32)
            h1b[:, sl] = jnp.maximum(
                acc + b1_ref[:, sl], 0.0).astype(jnp.bfloat16)
        layer2_and_out(h1b[...])


def _mlp3(x, w1, b1r, w2, b2r, w3, b3r, *, tm, tnc):
    M, K = x.shape
    N = w1.shape[1]
    O = w3.shape[1]
    nc = N // tnc
    nrows = M // tm
    nsteps = nc + (nrows - 1)
    flops = 2 * M * K * N + 2 * M * N * N + 2 * M * N * O
    bytes_accessed = 4 * (M * K + K * N + N * N + N * O + M * O)

    row_of = lambda s: jnp.maximum(s - (nc - 1), 0)
    chunk_of = lambda s: (0, jnp.minimum(s, nc - 1))
    return pl.pallas_call(
        _mlp3_kernel,
        out_shape=jax.ShapeDtypeStruct((M, O), jnp.float32),
        grid=(nsteps,),
        in_specs=[
            pl.BlockSpec((tm, K), lambda s: (row_of(s), 0)),
            pl.BlockSpec((K, tnc), lambda s: chunk_of(s)),
            pl.BlockSpec((1, N), lambda s: (0, 0)),
            pl.BlockSpec((K, tnc), lambda s: chunk_of(s)),
            pl.BlockSpec((1, N), lambda s: (0, 0)),
            pl.BlockSpec((N, O), lambda s: (0, 0)),
            pl.BlockSpec((1, O), lambda s: (0, 0)),
        ],
        out_specs=pl.BlockSpec((tm, O), lambda s: (row_of(s), 0)),
        scratch_shapes=[
            pltpu.VMEM((K, N), jnp.bfloat16),
            pltpu.VMEM((N, N), jnp.bfloat16),
            pltpu.VMEM((tm, K), jnp.bfloat16),
            pltpu.VMEM((tm, N), jnp.bfloat16),
            pltpu.VMEM((tm, N), jnp.bfloat16),
        ],
        compiler_params=pltpu.CompilerParams(
            dimension_semantics=("arbitrary",),
        ),
        cost_estimate=pl.CostEstimate(
            flops=flops, transcendentals=0, bytes_accessed=bytes_accessed
        ),
    )(x, w1, b1r, w2, b2r, w3, b3r)


@jax.jit
def kernel(x, w1, b1, w2, b2, w3, b3):
    M = x.shape[0]
    N = w1.shape[1]
    O = w3.shape[1]
    tm = min(512, max(M // 4, 8))
    tnc = min(512, max(N // 2, 128))
    return _mlp3(x, w1, b1.reshape(1, N), w2, b2.reshape(1, N),
                 w3, b3.reshape(1, O), tm=tm, tnc=tnc)


# speedup vs baseline: 1.1045x; 1.0305x over previous
import jax
import jax.numpy as jnp
from jax.experimental import pallas as pl
from jax.experimental.pallas import tpu as pltpu


def _mlp3_kernel(x_ref, w1_ref, b1_ref, w2_ref, b2_ref, w3_ref, b3_ref,
                 o_ref, w1b, w2b, h1b):
    n = b1_ref.shape[1]
    o = b3_ref.shape[1]
    tm = x_ref.shape[0]
    tnc = w1_ref.shape[1]
    nc = n // tnc
    s = pl.program_id(0)

    def layer2_and_out(h1src):
        w3c = w3_ref[...].astype(jnp.bfloat16)
        o_ref[...] = jnp.broadcast_to(b3_ref[...], (tm, o))
        for c in range(nc):
            sl = slice(c * tnc, (c + 1) * tnc)
            acc = jnp.dot(h1src, w2b[:, sl], preferred_element_type=jnp.float32)
            h2c = jnp.maximum(acc + b2_ref[:, sl], 0.0).astype(jnp.bfloat16)
            o_ref[...] += jnp.dot(h2c, w3c[sl, :],
                                  preferred_element_type=jnp.float32)

    @pl.when(s < nc)
    def _stream_chunks():
        sl = pl.ds(s * tnc, tnc)
        w1c = w1_ref[...].astype(jnp.bfloat16)
        w1b[:, sl] = w1c
        w2b[:, sl] = w2_ref[...].astype(jnp.bfloat16)
        xr = x_ref[...].astype(jnp.bfloat16)
        acc = jnp.dot(xr, w1c, preferred_element_type=jnp.float32)
        h1b[:, sl] = jnp.maximum(acc + b1_ref[:, sl], 0.0).astype(jnp.bfloat16)

    @pl.when(s == nc - 1)
    def _row0_tail():
        layer2_and_out(h1b[...])

    @pl.when(s >= nc)
    def _later_rows():
        xr = x_ref[...].astype(jnp.bfloat16)
        for c in range(nc):
            sl = slice(c * tnc, (c + 1) * tnc)
            acc = jnp.dot(xr, w1b[:, sl], preferred_element_type=jnp.float32)
            h1b[:, sl] = jnp.maximum(
                acc + b1_ref[:, sl], 0.0).astype(jnp.bfloat16)
        layer2_and_out(h1b[...])


def _mlp3(x, w1, b1r, w2, b2r, w3, b3r, *, tm, tnc):
    M, K = x.shape
    N = w1.shape[1]
    O = w3.shape[1]
    nc = N // tnc
    nrows = M // tm
    nsteps = nc + (nrows - 1)
    flops = 2 * M * K * N + 2 * M * N * N + 2 * M * N * O
    bytes_accessed = 4 * (M * K + K * N + N * N + N * O + M * O)

    row_of = lambda s: jnp.maximum(s - (nc - 1), 0)
    chunk_of = lambda s: (0, jnp.minimum(s, nc - 1))
    return pl.pallas_call(
        _mlp3_kernel,
        out_shape=jax.ShapeDtypeStruct((M, O), jnp.float32),
        grid=(nsteps,),
        in_specs=[
            pl.BlockSpec((tm, K), lambda s: (row_of(s), 0)),
            pl.BlockSpec((K, tnc), lambda s: chunk_of(s)),
            pl.BlockSpec((1, N), lambda s: (0, 0)),
            pl.BlockSpec((K, tnc), lambda s: chunk_of(s)),
            pl.BlockSpec((1, N), lambda s: (0, 0)),
            pl.BlockSpec((N, O), lambda s: (0, 0)),
            pl.BlockSpec((1, O), lambda s: (0, 0)),
        ],
        out_specs=pl.BlockSpec((tm, O), lambda s: (row_of(s), 0)),
        scratch_shapes=[
            pltpu.VMEM((K, N), jnp.bfloat16),
            pltpu.VMEM((N, N), jnp.bfloat16),
            pltpu.VMEM((tm, N), jnp.bfloat16),
        ],
        compiler_params=pltpu.CompilerParams(
            dimension_semantics=("arbitrary",),
        ),
        cost_estimate=pl.CostEstimate(
            flops=flops, transcendentals=0, bytes_accessed=bytes_accessed
        ),
    )(x, w1, b1r, w2, b2r, w3, b3r)


@jax.jit
def kernel(x, w1, b1, w2, b2, w3, b3):
    M = x.shape[0]
    N = w1.shape[1]
    O = w3.shape[1]
    tm = min(1024, max(M // 2, 8))
    tnc = min(512, max(N // 2, 128))
    return _mlp3(x, w1, b1.reshape(1, N), w2, b2.reshape(1, N),
                 w3, b3.reshape(1, O), tm=tm, tnc=tnc)


# speedup vs baseline: 1.1068x; 1.0021x over previous
import jax
import jax.numpy as jnp
from jax.experimental import pallas as pl
from jax.experimental.pallas import tpu as pltpu


def _mlp3_kernel(x_ref, w1_ref, b1_ref, w2_ref, b2_ref, w3_ref, b3_ref,
                 o_ref, w1b, w2b, h1b):
    n = b1_ref.shape[1]
    o = b3_ref.shape[1]
    tm = x_ref.shape[0]
    tnc = w1_ref.shape[1]
    nc = n // tnc
    s = pl.program_id(0)

    def layer2_and_out(h1src):
        w3c = w3_ref[...].astype(jnp.bfloat16)
        o_ref[...] = jnp.broadcast_to(b3_ref[...], (tm, o))
        for c in range(nc):
            sl = slice(c * tnc, (c + 1) * tnc)
            acc = jnp.dot(h1src, w2b[:, sl], preferred_element_type=jnp.float32)
            h2c = jnp.maximum(acc + b2_ref[:, sl], 0.0).astype(jnp.bfloat16)
            o_ref[...] += jnp.dot(h2c, w3c[sl, :],
                                  preferred_element_type=jnp.float32)

    @pl.when(s < nc)
    def _stream_chunks():
        sl = pl.ds(s * tnc, tnc)
        w1c = w1_ref[...].astype(jnp.bfloat16)
        w1b[:, sl] = w1c
        w2b[:, sl] = w2_ref[...].astype(jnp.bfloat16)
        xr = x_ref[...].astype(jnp.bfloat16)
        acc = jnp.dot(xr, w1c, preferred_element_type=jnp.float32)
        h1b[:, sl] = jnp.maximum(acc + b1_ref[:, sl], 0.0).astype(jnp.bfloat16)

    @pl.when(s == nc - 1)
    def _row0_tail():
        layer2_and_out(h1b[...])

    @pl.when(s >= nc)
    def _later_rows():
        wide = min(1024, n)
        xr = x_ref[...].astype(jnp.bfloat16)
        for c in range(n // wide):
            sl = slice(c * wide, (c + 1) * wide)
            acc = jnp.dot(xr, w1b[:, sl], preferred_element_type=jnp.float32)
            h1b[:, sl] = jnp.maximum(
                acc + b1_ref[:, sl], 0.0).astype(jnp.bfloat16)
        layer2_and_out(h1b[...])


def _mlp3(x, w1, b1r, w2, b2r, w3, b3r, *, tm, tnc):
    M, K = x.shape
    N = w1.shape[1]
    O = w3.shape[1]
    nc = N // tnc
    nrows = M // tm
    nsteps = nc + (nrows - 1)
    flops = 2 * M * K * N + 2 * M * N * N + 2 * M * N * O
    bytes_accessed = 4 * (M * K + K * N + N * N + N * O + M * O)

    row_of = lambda s: jnp.maximum(s - (nc - 1), 0)
    chunk_of = lambda s: (0, jnp.minimum(s, nc - 1))
    return pl.pallas_call(
        _mlp3_kernel,
        out_shape=jax.ShapeDtypeStruct((M, O), jnp.float32),
        grid=(nsteps,),
        in_specs=[
            pl.BlockSpec((tm, K), lambda s: (row_of(s), 0)),
            pl.BlockSpec((K, tnc), lambda s: chunk_of(s)),
            pl.BlockSpec((1, N), lambda s: (0, 0)),
            pl.BlockSpec((K, tnc), lambda s: chunk_of(s)),
            pl.BlockSpec((1, N), lambda s: (0, 0)),
            pl.BlockSpec((N, O), lambda s: (0, 0)),
            pl.BlockSpec((1, O), lambda s: (0, 0)),
        ],
        out_specs=pl.BlockSpec((tm, O), lambda s: (row_of(s), 0)),
        scratch_shapes=[
            pltpu.VMEM((K, N), jnp.bfloat16),
            pltpu.VMEM((N, N), jnp.bfloat16),
            pltpu.VMEM((tm, N), jnp.bfloat16),
        ],
        compiler_params=pltpu.CompilerParams(
            dimension_semantics=("arbitrary",),
        ),
        cost_estimate=pl.CostEstimate(
            flops=flops, transcendentals=0, bytes_accessed=bytes_accessed
        ),
    )(x, w1, b1r, w2, b2r, w3, b3r)


@jax.jit
def kernel(x, w1, b1, w2, b2, w3, b3):
    M = x.shape[0]
    N = w1.shape[1]
    O = w3.shape[1]
    tm = min(1024, max(M // 2, 8))
    tnc = min(512, max(N // 2, 128))
    return _mlp3(x, w1, b1.reshape(1, N), w2, b2.reshape(1, N),
                 w3, b3.reshape(1, O), tm=tm, tnc=tnc)
